# Initial kernel scaffold; baseline (speedup 1.0000x reference)
import jax
import jax.numpy as jnp
from jax.experimental import pallas as pl


def kernel(latent_amino, latent_pep, peptides, alleles, lengths, pretrain, actions, W_pos, b_pos, W_amino, b_amino):
    raise NotImplementedError("write your pallas kernel here")


# speedup vs baseline: 2.1276x; 2.1276x over previous
import jax
import jax.numpy as jnp
from jax.experimental import pallas as pl

_NEG = -100000.0


def _body(lat_ref, len_ref, pos_ref, pep_ref, wpos_ref, bpos_ref,
          wam_ref, bam_ref, out_pos_ref, out_am_ref):
    T, BB, D = lat_ref.shape
    w = wpos_ref[0, :]
    pos_ac = pos_ref[...]
    lens = len_ref[...]
    cols = []
    acc = jnp.zeros((BB, D), dtype=jnp.float32)
    for t in range(T):
        x_t = lat_ref[t]
        s_t = jnp.sum(x_t * w[None, :], axis=1, keepdims=True)
        cols.append(s_t)
        acc = acc + jnp.where(pos_ac == t, x_t, 0.0)
    scores = jnp.concatenate(cols, axis=1) + bpos_ref[0, 0]
    t_iota = jax.lax.broadcasted_iota(jnp.int32, (BB, T), 1)
    out_pos_ref[...] = jnp.where(t_iota < lens, scores, _NEG)

    am = jax.lax.dot_general(acc, wam_ref[...], (((1,), (1,)), ((), ())),
                             preferred_element_type=jnp.float32)
    am = am + bam_ref[...]
    pep = pep_ref[...]
    pep_sel = jnp.sum(jnp.where(t_iota == pos_ac, pep, 0), axis=1,
                      keepdims=True)
    mask_col = jnp.where(pep_sel == 0, 19, pep_sel - 1)
    k_iota = jax.lax.broadcasted_iota(jnp.int32, (BB, 20), 1)
    out_am_ref[...] = jnp.where(k_iota == mask_col, _NEG, am)


def kernel(latent_amino, latent_pep, peptides, alleles, lengths, pretrain,
           actions, W_pos, b_pos, W_amino, b_amino):
    T, B, D = latent_amino.shape
    BB = 512
    lengths2 = lengths.astype(jnp.int32).reshape(B, 1)
    pos_ac = actions[:, 0:1].astype(jnp.int32)
    pep = peptides.astype(jnp.int32)
    bpos2 = b_pos.reshape(1, 1).astype(jnp.float32)
    bam2 = b_amino.reshape(1, -1).astype(jnp.float32)
    f = pl.pallas_call(
        _body,
        grid=(B // BB,),
        in_specs=[
            pl.BlockSpec((T, BB, D), lambda i: (0, i, 0)),
            pl.BlockSpec((BB, 1), lambda i: (i, 0)),
            pl.BlockSpec((BB, 1), lambda i: (i, 0)),
            pl.BlockSpec((BB, T), lambda i: (i, 0)),
            pl.BlockSpec((1, D), lambda i: (0, 0)),
            pl.BlockSpec((1, 1), lambda i: (0, 0)),
            pl.BlockSpec((20, D), lambda i: (0, 0)),
            pl.BlockSpec((1, 20), lambda i: (0, 0)),
        ],
        out_specs=(
            pl.BlockSpec((BB, T), lambda i: (i, 0)),
            pl.BlockSpec((BB, 20), lambda i: (i, 0)),
        ),
        out_shape=(
            jax.ShapeDtypeStruct((B, T), jnp.float32),
            jax.ShapeDtypeStruct((B, 20), jnp.float32),
        ),
    )
    return f(latent_amino, lengths2, pos_ac, pep, W_pos, bpos2, W_amino, bam2)


# speedup vs baseline: 3.4289x; 1.6116x over previous
import jax
import jax.numpy as jnp
from jax.experimental import pallas as pl

_NEG = -100000.0


def _body(lat_ref, len_ref, pos_ref, pep_ref, wpos_ref, bpos_ref,
          wam_ref, bam_ref, out_pos_ref, out_am_ref):
    T, BB, D = lat_ref.shape
    w_row = wpos_ref[...]
    pos_ac = pos_ref[...]
    lens_row = len_ref[...]
    rows = []
    acc = jnp.zeros((BB, D), dtype=jnp.float32)
    for t in range(T):
        x_t = lat_ref[t]
        s_t = jax.lax.dot_general(w_row, x_t, (((1,), (1,)), ((), ())),
                                  preferred_element_type=jnp.float32)
        rows.append(s_t)
        m_t = (pos_ac == t).astype(jnp.float32)
        acc = acc + m_t * x_t
    scores_T = jnp.concatenate(rows, axis=0) + bpos_ref[0, 0]
    ti = jax.lax.broadcasted_iota(jnp.int32, (T, BB), 0)
    out_pos_ref[...] = jnp.where(ti < lens_row, scores_T, _NEG)

    am = jax.lax.dot_general(acc, wam_ref[...], (((1,), (1,)), ((), ())),
                             preferred_element_type=jnp.float32)
    am = am + bam_ref[...]
    pep = pep_ref[...]
    lane_t = jax.lax.broadcasted_iota(jnp.int32, (BB, T), 1)
    pep_sel = jnp.sum(jnp.where(lane_t == pos_ac, pep, 0), axis=1,
                      keepdims=True)
    mask_col = jnp.where(pep_sel == 0, 19, pep_sel - 1)
    k_iota = jax.lax.broadcasted_iota(jnp.int32, (BB, 20), 1)
    out_am_ref[...] = jnp.where(k_iota == mask_col, _NEG, am)


def kernel(latent_amino, latent_pep, peptides, alleles, lengths, pretrain,
           actions, W_pos, b_pos, W_amino, b_amino):
    T, B, D = latent_amino.shape
    BB = 512
    lengths2 = lengths.astype(jnp.int32).reshape(1, B)
    pos_ac = actions[:, 0:1].astype(jnp.int32)
    pep = peptides.astype(jnp.int32)
    bpos2 = b_pos.reshape(1, 1).astype(jnp.float32)
    bam2 = b_amino.reshape(1, -1).astype(jnp.float32)
    f = pl.pallas_call(
        _body,
        grid=(B // BB,),
        in_specs=[
            pl.BlockSpec((T, BB, D), lambda i: (0, i, 0)),
            pl.BlockSpec((1, BB), lambda i: (0, i)),
            pl.BlockSpec((BB, 1), lambda i: (i, 0)),
            pl.BlockSpec((BB, T), lambda i: (i, 0)),
            pl.BlockSpec((1, D), lambda i: (0, 0)),
            pl.BlockSpec((1, 1), lambda i: (0, 0)),
            pl.BlockSpec((20, D), lambda i: (0, 0)),
            pl.BlockSpec((1, 20), lambda i: (0, 0)),
        ],
        out_specs=(
            pl.BlockSpec((T, BB), lambda i: (0, i)),
            pl.BlockSpec((BB, 20), lambda i: (i, 0)),
        ),
        out_shape=(
            jax.ShapeDtypeStruct((T, B), jnp.float32),
            jax.ShapeDtypeStruct((B, 20), jnp.float32),
        ),
    )
    scores_T, amino_pd = f(latent_amino, lengths2, pos_ac, pep, W_pos,
                           bpos2, W_amino, bam2)
    return (scores_T.T, amino_pd)


# speedup vs baseline: 3.6532x; 1.0654x over previous
import jax
import jax.numpy as jnp
from jax.experimental import pallas as pl

_NEG = -100000.0


def _body(lat_ref, len_ref, pos_ref, pep_ref, wpos_ref, bpos_ref,
          wam_ref, bam_ref, out_pos_ref, out_am_ref):
    T, BB, D = lat_ref.shape
    w_row = wpos_ref[...]
    pos_ac = pos_ref[...]
    lens_row = len_ref[...]
    rows = []
    acc = jnp.zeros((BB, D), dtype=jnp.float32)
    for t in range(T):
        x_t = lat_ref[t]
        s_t = jax.lax.dot_general(w_row, x_t, (((1,), (1,)), ((), ())),
                                  preferred_element_type=jnp.float32)
        rows.append(s_t)
        m_t = (pos_ac == t).astype(jnp.float32)
        acc = acc + m_t * x_t
    scores_T = jnp.concatenate(rows, axis=0) + bpos_ref[0, 0]
    ti = jax.lax.broadcasted_iota(jnp.int32, (T, BB), 0)
    out_pos_ref[...] = jnp.where(ti < lens_row, scores_T, _NEG)

    am = jax.lax.dot_general(acc, wam_ref[...], (((1,), (1,)), ((), ())),
                             preferred_element_type=jnp.float32)
    am = am + bam_ref[...]
    pep = pep_ref[...]
    lane_t = jax.lax.broadcasted_iota(jnp.int32, (BB, T), 1)
    pep_sel = jnp.sum(jnp.where(lane_t == pos_ac, pep, 0), axis=1,
                      keepdims=True)
    mask_col = jnp.where(pep_sel == 0, 19, pep_sel - 1)
    k_iota = jax.lax.broadcasted_iota(jnp.int32, (BB, 20), 1)
    out_am_ref[...] = jnp.where(k_iota == mask_col, _NEG, am)


def kernel(latent_amino, latent_pep, peptides, alleles, lengths, pretrain,
           actions, W_pos, b_pos, W_amino, b_amino):
    T, B, D = latent_amino.shape
    BB = 1024
    lengths2 = lengths.astype(jnp.int32).reshape(1, B)
    pos_ac = actions[:, 0:1].astype(jnp.int32)
    pep = peptides.astype(jnp.int32)
    bpos2 = b_pos.reshape(1, 1).astype(jnp.float32)
    bam2 = b_amino.reshape(1, -1).astype(jnp.float32)
    f = pl.pallas_call(
        _body,
        grid=(B // BB,),
        in_specs=[
            pl.BlockSpec((T, BB, D), lambda i: (0, i, 0)),
            pl.BlockSpec((1, BB), lambda i: (0, i)),
            pl.BlockSpec((BB, 1), lambda i: (i, 0)),
            pl.BlockSpec((BB, T), lambda i: (i, 0)),
            pl.BlockSpec((1, D), lambda i: (0, 0)),
            pl.BlockSpec((1, 1), lambda i: (0, 0)),
            pl.BlockSpec((20, D), lambda i: (0, 0)),
            pl.BlockSpec((1, 20), lambda i: (0, 0)),
        ],
        out_specs=(
            pl.BlockSpec((T, BB), lambda i: (0, i)),
            pl.BlockSpec((BB, 20), lambda i: (i, 0)),
        ),
        out_shape=(
            jax.ShapeDtypeStruct((T, B), jnp.float32),
            jax.ShapeDtypeStruct((B, 20), jnp.float32),
        ),
    )
    scores_T, amino_pd = f(latent_amino, lengths2, pos_ac, pep, W_pos,
                           bpos2, W_amino, bam2)
    return (scores_T.T, amino_pd)
